# baseline (speedup 1.0000x reference)
import jax
import jax.numpy as jnp
from jax import lax
from jax.experimental import pallas as pl

_L = 128


def _cd_kernel(q_ref, dbc_ref, dist_ref, idx_ref):
    s = q_ref.shape[3]
    g = dbc_ref.shape[3]

    qx = q_ref[0, 0, 0]
    qy = q_ref[0, 0, 1]
    qz = q_ref[0, 0, 2]

    inf = jnp.full((s, _L), jnp.inf, jnp.float32)
    zero = jnp.zeros((s, _L), jnp.float32)
    lane = jax.lax.broadcasted_iota(jnp.int32, (1, _L), 1).astype(jnp.float32)

    def body(j, carry):
        rmin, ridx = carry
        cx = dbc_ref[0, 0, 0, pl.ds(j, 1), :]
        cy = dbc_ref[0, 0, 1, pl.ds(j, 1), :]
        cz = dbc_ref[0, 0, 2, pl.ds(j, 1), :]
        ivrow = lane + (j * _L).astype(jnp.float32)
        for k in range(_L):
            dx = qx - cx[:, k:k + 1]
            dy = qy - cy[:, k:k + 1]
            dz = qz - cz[:, k:k + 1]
            d = dx * dx + dy * dy + dz * dz
            take = d < rmin
            rmin = jnp.where(take, d, rmin)
            ridx = jnp.where(take, ivrow[:, k:k + 1], ridx)
        return rmin, ridx

    rmin, ridx = lax.fori_loop(0, g, body, (inf, zero))
    dist_ref[0, 0] = rmin
    idx_ref[0, 0] = ridx.astype(jnp.int32)


def _chamfer_both(q, dbc):
    _, b, _, s, _ = q.shape
    g = dbc.shape[3]
    dist, idx = pl.pallas_call(
        _cd_kernel,
        grid=(2, b),
        in_specs=[
            pl.BlockSpec((1, 1, 3, s, _L), lambda p, bi: (p, bi, 0, 0, 0)),
            pl.BlockSpec((1, 1, 3, g, _L), lambda p, bi: (p, bi, 0, 0, 0)),
        ],
        out_specs=[
            pl.BlockSpec((1, 1, s, _L), lambda p, bi: (p, bi, 0, 0)),
            pl.BlockSpec((1, 1, s, _L), lambda p, bi: (p, bi, 0, 0)),
        ],
        out_shape=[
            jax.ShapeDtypeStruct((2, b, s, _L), jnp.float32),
            jax.ShapeDtypeStruct((2, b, s, _L), jnp.int32),
        ],
    )(q, dbc)
    return dist, idx


def kernel(input1, input2):
    b, n, _ = input1.shape
    s = n // _L
    g = n // _L
    x1t = jnp.transpose(input1, (0, 2, 1))
    x2t = jnp.transpose(input2, (0, 2, 1))
    q = jnp.stack([x1t.reshape(b, 3, s, _L), x2t.reshape(b, 3, s, _L)])
    dbc = jnp.stack([x2t, x1t]).reshape(2, b, 3, g, _L)
    dist, idx = _chamfer_both(q, dbc)
    dist1 = dist[0].reshape(b, n)
    idx1 = idx[0].reshape(b, n)
    dist2 = dist[1].reshape(b, n)
    idx2 = idx[1].reshape(b, n)
    return (dist1, idx1, dist2, idx2)


# speedup vs baseline: 1.1177x; 1.1177x over previous
import jax
import jax.numpy as jnp
from jax import lax
from jax.experimental import pallas as pl

_U = 64
_L = 128


def _cd_kernel(q_ref, dbp_ref, iv_ref, dist_ref, idx_ref):
    s = q_ref.shape[3]
    g = dbp_ref.shape[2]

    qx = q_ref[0, 0, 0]
    qy = q_ref[0, 0, 1]
    qz = q_ref[0, 0, 2]

    inf = jnp.full((s, _L), jnp.inf, jnp.float32)
    zero = jnp.zeros((s, _L), jnp.float32)

    def body(j, carry):
        rmin, ridx = carry
        blk = dbp_ref[0, 0, pl.ds(j, 1)]
        ivb = iv_ref[pl.ds(j * _U, _U), :]
        for k in range(_U):
            dx = qx - blk[0, k:k + 1, :]
            dy = qy - blk[0, _U + k:_U + k + 1, :]
            dz = qz - blk[0, 2 * _U + k:2 * _U + k + 1, :]
            d = dx * dx + dy * dy + dz * dz
            take = d < rmin
            rmin = jnp.where(take, d, rmin)
            ridx = jnp.where(take, ivb[k:k + 1, :], ridx)
        return rmin, ridx

    rmin, ridx = lax.fori_loop(0, g, body, (inf, zero))
    dist_ref[0, 0] = rmin
    idx_ref[0, 0] = ridx.astype(jnp.int32)


def _chamfer_both(q, dbp, iv):
    _, b, _, s, _ = q.shape
    g = dbp.shape[2]
    m = iv.shape[0]
    dist, idx = pl.pallas_call(
        _cd_kernel,
        grid=(2, b),
        in_specs=[
            pl.BlockSpec((1, 1, 3, s, _L), lambda p, bi: (p, bi, 0, 0, 0)),
            pl.BlockSpec((1, 1, g, 3 * _U, _L),
                         lambda p, bi: (p, bi, 0, 0, 0)),
            pl.BlockSpec((m, _L), lambda p, bi: (0, 0)),
        ],
        out_specs=[
            pl.BlockSpec((1, 1, s, _L), lambda p, bi: (p, bi, 0, 0)),
            pl.BlockSpec((1, 1, s, _L), lambda p, bi: (p, bi, 0, 0)),
        ],
        out_shape=[
            jax.ShapeDtypeStruct((2, b, s, _L), jnp.float32),
            jax.ShapeDtypeStruct((2, b, s, _L), jnp.int32),
        ],
    )(q, dbp, iv)
    return dist, idx


def kernel(input1, input2):
    b, n, _ = input1.shape
    s = n // _L
    g = n // _U
    x1t = jnp.transpose(input1, (0, 2, 1))
    x2t = jnp.transpose(input2, (0, 2, 1))
    q = jnp.stack([x1t.reshape(b, 3, s, _L), x2t.reshape(b, 3, s, _L)])
    db = jnp.stack([x2t, x1t])
    dbp = db.reshape(2, b, 3, g, _U).transpose(0, 1, 3, 2, 4)
    dbp = dbp.reshape(2, b, g, 3 * _U)
    dbp = jnp.broadcast_to(dbp[..., None], dbp.shape + (_L,))
    iv = jnp.broadcast_to(
        jnp.arange(n, dtype=jnp.float32)[:, None], (n, _L))
    dist, idx = _chamfer_both(q, dbp, iv)
    dist1 = dist[0].reshape(b, n)
    idx1 = idx[0].reshape(b, n)
    dist2 = dist[1].reshape(b, n)
    idx2 = idx[1].reshape(b, n)
    return (dist1, idx1, dist2, idx2)


# speedup vs baseline: 1.1277x; 1.0090x over previous
import jax
import jax.numpy as jnp
from jax import lax
from jax.experimental import pallas as pl

_U = 128
_L = 128


def _cd_kernel(q_ref, dbp_ref, iv_ref, dist_ref, idx_ref):
    s = q_ref.shape[3]
    g = dbp_ref.shape[2]

    qx = q_ref[0, 0, 0]
    qy = q_ref[0, 0, 1]
    qz = q_ref[0, 0, 2]

    inf = jnp.full((s, _L), jnp.inf, jnp.float32)
    zero = jnp.zeros((s, _L), jnp.float32)

    def body(j, carry):
        rmin, ridx = carry
        blk = dbp_ref[0, 0, pl.ds(j, 1)]
        ivb = iv_ref[pl.ds(j * _U, _U), :]
        for k in range(_U):
            dx = qx - blk[0, k:k + 1, :]
            dy = qy - blk[0, _U + k:_U + k + 1, :]
            dz = qz - blk[0, 2 * _U + k:2 * _U + k + 1, :]
            d = dx * dx + dy * dy + dz * dz
            take = d < rmin
            rmin = jnp.where(take, d, rmin)
            ridx = jnp.where(take, ivb[k:k + 1, :], ridx)
        return rmin, ridx

    rmin, ridx = lax.fori_loop(0, g, body, (inf, zero))
    dist_ref[0, 0] = rmin
    idx_ref[0, 0] = ridx.astype(jnp.int32)


def _chamfer_both(q, dbp, iv):
    _, b, _, s, _ = q.shape
    g = dbp.shape[2]
    m = iv.shape[0]
    dist, idx = pl.pallas_call(
        _cd_kernel,
        grid=(2, b),
        in_specs=[
            pl.BlockSpec((1, 1, 3, s, _L), lambda p, bi: (p, bi, 0, 0, 0)),
            pl.BlockSpec((1, 1, g, 3 * _U, _L),
                         lambda p, bi: (p, bi, 0, 0, 0)),
            pl.BlockSpec((m, _L), lambda p, bi: (0, 0)),
        ],
        out_specs=[
            pl.BlockSpec((1, 1, s, _L), lambda p, bi: (p, bi, 0, 0)),
            pl.BlockSpec((1, 1, s, _L), lambda p, bi: (p, bi, 0, 0)),
        ],
        out_shape=[
            jax.ShapeDtypeStruct((2, b, s, _L), jnp.float32),
            jax.ShapeDtypeStruct((2, b, s, _L), jnp.int32),
        ],
    )(q, dbp, iv)
    return dist, idx


def kernel(input1, input2):
    b, n, _ = input1.shape
    s = n // _L
    g = n // _U
    x1t = jnp.transpose(input1, (0, 2, 1))
    x2t = jnp.transpose(input2, (0, 2, 1))
    q = jnp.stack([x1t.reshape(b, 3, s, _L), x2t.reshape(b, 3, s, _L)])
    db = jnp.stack([x2t, x1t])
    dbp = db.reshape(2, b, 3, g, _U).transpose(0, 1, 3, 2, 4)
    dbp = dbp.reshape(2, b, g, 3 * _U)
    dbp = jnp.broadcast_to(dbp[..., None], dbp.shape + (_L,))
    iv = jnp.broadcast_to(
        jnp.arange(n, dtype=jnp.float32)[:, None], (n, _L))
    dist, idx = _chamfer_both(q, dbp, iv)
    dist1 = dist[0].reshape(b, n)
    idx1 = idx[0].reshape(b, n)
    dist2 = dist[1].reshape(b, n)
    idx2 = idx[1].reshape(b, n)
    return (dist1, idx1, dist2, idx2)
